# Initial kernel scaffold; baseline (speedup 1.0000x reference)
import jax
import jax.numpy as jnp
from jax.experimental import pallas as pl


def kernel(xyz):
    raise NotImplementedError("write your pallas kernel here")


# speedup vs baseline: 11.7422x; 11.7422x over previous
import functools

import jax
import jax.numpy as jnp
from jax import lax
from jax.experimental import pallas as pl
from jax.experimental.pallas import tpu as pltpu
from jax.experimental.pallas import tpu_sc as plsc

B = 16
N = 8192
G = 512
M = 32
GC = 128
NCHUNK = G // GC

D_PAD = 16
ROWS_TOTAL = B * G * M
NW = 32
ROWS_PER_W = ROWS_TOTAL // NW
CH = 2048


def _fps_body(x_ref, y_ref, z_ref, cx_ref, cy_ref, cz_ref):
    x = x_ref[:, 0, :]
    y = y_ref[:, 0, :]
    z = z_ref[:, 0, :]
    lane = lax.broadcasted_iota(jnp.int32, (B, N), 1)
    col = lax.broadcasted_iota(jnp.int32, (B, G), 1)

    def body(i, st):
        distance, farthest, cxs, cys, czs = st
        sel = lane == farthest[:, None]
        cx = jnp.sum(jnp.where(sel, x, 0.0), axis=1)
        cy = jnp.sum(jnp.where(sel, y, 0.0), axis=1)
        cz = jnp.sum(jnp.where(sel, z, 0.0), axis=1)
        cxs = jnp.where(col == i, cx[:, None], cxs)
        cys = jnp.where(col == i, cy[:, None], cys)
        czs = jnp.where(col == i, cz[:, None], czs)
        dx = x - cx[:, None]
        dy = y - cy[:, None]
        dz = z - cz[:, None]
        dist = (dx * dx + dy * dy) + dz * dz
        distance = jnp.minimum(distance, dist)
        farthest = jnp.argmax(distance, axis=1).astype(jnp.int32)
        return (distance, farthest, cxs, cys, czs)

    init = (
        jnp.full((B, N), 1e10, dtype=jnp.float32),
        jnp.zeros((B,), dtype=jnp.int32),
        jnp.zeros((B, G), dtype=jnp.float32),
        jnp.zeros((B, G), dtype=jnp.float32),
        jnp.zeros((B, G), dtype=jnp.float32),
    )
    _, _, cxs, cys, czs = lax.fori_loop(0, G, body, init)
    cx_ref[...] = cxs
    cy_ref[...] = cys
    cz_ref[...] = czs


def _fps_call(x, y, z):
    return pl.pallas_call(
        _fps_body,
        out_shape=[jax.ShapeDtypeStruct((B, G), jnp.float32)] * 3,
    )(x, y, z)


def _knn_body(x_ref, y_ref, z_ref, cx_ref, cy_ref, cz_ref, idx_ref):
    b = pl.program_id(0)
    px = x_ref[0]
    py = y_ref[0]
    pz = z_ref[0]
    cx = cx_ref[0, 0]
    cy = cy_ref[0, 0]
    cz = cz_ref[0, 0]

    def bf(v):
        return v.astype(jnp.bfloat16).astype(jnp.float32)

    dot = (bf(cx) * bf(px) + bf(cy) * bf(py)) + bf(cz) * bf(pz)
    pn = (px * px + py * py) + pz * pz
    cn = (cx * cx + cy * cy) + cz * cz
    dist = ((-2.0) * dot + cn) + pn

    lane = lax.broadcasted_iota(jnp.int32, (GC, N), 1)
    kcol = lax.broadcasted_iota(jnp.int32, (GC, M), 1)
    acc = jnp.zeros((GC, M), dtype=jnp.int32)
    for k in range(M):
        j = jnp.argmin(dist, axis=1).astype(jnp.int32)
        acc = jnp.where(kcol == k, j[:, None], acc)
        dist = jnp.where(lane == j[:, None], jnp.inf, dist)
    idx_ref[0, 0] = acc + b * N


def _knn_call(x, y, z, cxr, cyr, czr):
    return pl.pallas_call(
        _knn_body,
        grid=(B, NCHUNK),
        in_specs=[
            pl.BlockSpec((1, 1, N), lambda b, c: (b, 0, 0)),
            pl.BlockSpec((1, 1, N), lambda b, c: (b, 0, 0)),
            pl.BlockSpec((1, 1, N), lambda b, c: (b, 0, 0)),
            pl.BlockSpec((1, 1, GC, 1), lambda b, c: (b, c, 0, 0)),
            pl.BlockSpec((1, 1, GC, 1), lambda b, c: (b, c, 0, 0)),
            pl.BlockSpec((1, 1, GC, 1), lambda b, c: (b, c, 0, 0)),
        ],
        out_specs=pl.BlockSpec((1, 1, GC, M), lambda b, c: (b, c, 0, 0)),
        out_shape=jax.ShapeDtypeStruct((B, NCHUNK, GC, M), jnp.int32),
    )(x, y, z, cxr, cyr, czr)


def _sc_gather_kernel(table, ctable, idx, cidx, out,
                      idx_v, cidx_v, rows_v, crows_v, sem1, sem2):
    wid = lax.axis_index("s") * 2 + lax.axis_index("c")
    for c in range(ROWS_PER_W // CH):
        base = wid * ROWS_PER_W + c * CH
        pltpu.sync_copy(idx.at[pl.ds(base, CH)], idx_v)
        pltpu.sync_copy(cidx.at[pl.ds(base, CH)], cidx_v)
        cp1 = pltpu.async_copy(table.at[idx_v], rows_v, sem1)
        cp2 = pltpu.async_copy(ctable.at[cidx_v], crows_v, sem2)
        cp1.wait()
        cp2.wait()

        def sub_body(i, _):
            for u in range(4):
                r = i * 4 + u
                rows_v[r] = rows_v[r] - crows_v[r]
            return 0

        lax.fori_loop(0, CH // 4, sub_body, 0)
        pltpu.sync_copy(rows_v, out.at[pl.ds(base, CH)])


@functools.partial(
    pl.kernel,
    mesh=plsc.VectorSubcoreMesh(core_axis_name="c", subcore_axis_name="s"),
    compiler_params=pltpu.CompilerParams(use_tc_tiling_on_sc=False),
    out_type=jax.ShapeDtypeStruct((ROWS_TOTAL, D_PAD), jnp.float32),
    scratch_types=[
        pltpu.VMEM((CH,), jnp.int32),
        pltpu.VMEM((CH,), jnp.int32),
        pltpu.VMEM((CH, D_PAD), jnp.float32),
        pltpu.VMEM((CH, D_PAD), jnp.float32),
        pltpu.SemaphoreType.DMA,
        pltpu.SemaphoreType.DMA,
    ],
)
def _sc_gather(table, ctable, idx, cidx, out,
               idx_v, cidx_v, rows_v, crows_v, sem1, sem2):
    _sc_gather_kernel(table, ctable, idx, cidx, out,
                      idx_v, cidx_v, rows_v, crows_v, sem1, sem2)


def kernel(xyz):
    x = xyz[:, :, 0].reshape(B, 1, N)
    y = xyz[:, :, 1].reshape(B, 1, N)
    z = xyz[:, :, 2].reshape(B, 1, N)

    cx, cy, cz = _fps_call(x, y, z)
    center = jnp.stack([cx, cy, cz], axis=-1)

    cxr = cx.reshape(B, NCHUNK, GC, 1)
    cyr = cy.reshape(B, NCHUNK, GC, 1)
    czr = cz.reshape(B, NCHUNK, GC, 1)
    flat_idx = _knn_call(x, y, z, cxr, cyr, czr).reshape(-1)

    table = jnp.pad(xyz.reshape(B * N, 3), ((0, 0), (0, D_PAD - 3)))
    ctab = jnp.pad(center.reshape(B * G, 3), ((0, 0), (0, D_PAD - 3)))
    cidx = jnp.arange(ROWS_TOTAL, dtype=jnp.int32) // M

    nb = _sc_gather(table, ctab, flat_idx, cidx)
    neighborhood = nb.reshape(B, G, M, D_PAD)[..., :3]
    return (neighborhood, center)
